# baseline (speedup 1.0000x reference)
import functools

import jax
import jax.numpy as jnp
import numpy as np
from jax import lax
from jax.experimental import pallas as pl
from jax.experimental.pallas import tpu as pltpu
from jax.experimental.pallas import tpu_sc as plsc

H = 224
W = 224
L = 16
NV = W // L
NC = 2
NS = 16
NW = NC * NS
RPW = 8
NWORK = H // RPW
SEG = 8

INF = np.float32(np.inf)
BIG = np.float32(1e30)


def _newton_sqrt(x):
    xc = jnp.where(x < BIG, jnp.maximum(x, np.float32(1.0)), np.float32(1.0))
    xr = xc
    scale = jnp.full_like(x, np.float32(1.0))
    for p in range(8, 0, -1):
        c = xr >= np.float32(4.0**p)
        xr = jnp.where(c, xr * np.float32(4.0 ** (-p)), xr)
        scale = jnp.where(c, scale * np.float32(2.0**p), scale)
    half = np.float32(0.5)
    y = half * (xr + np.float32(1.0))
    for _ in range(3):
        y = half * (y + xr / y)
    s = scale * y
    s = jnp.where(x < BIG, s, INF)
    return jnp.where(x == np.float32(0.0), np.float32(0.0), s)


def _make_edt():
    mesh = plsc.VectorSubcoreMesh(
        core_axis_name="c", subcore_axis_name="s",
        num_cores=NC, num_subcores=NS,
    )

    @functools.partial(
        pl.kernel,
        out_type=jax.ShapeDtypeStruct((H, W), jnp.float32),
        mesh=mesh,
        scratch_types=[
            pltpu.VMEM((H, W), jnp.float32),
            pltpu.VMEM((4, W), jnp.float32),
            pltpu.VMEM((RPW, W), jnp.float32),
            pltpu.VMEM((RPW, W), jnp.float32),
        ],
    )
    def edt(fm_hbm, out_hbm, fm_v, ab_v, g2_v, out_v):
        wid = lax.axis_index("s") * NC + lax.axis_index("c")
        r0 = wid * RPW

        @pl.when(wid < NWORK)
        def _active():
            _edt_body(fm_hbm, out_hbm, fm_v, ab_v, g2_v, out_v, wid, r0)

    def _edt_body(fm_hbm, out_hbm, fm_v, ab_v, g2_v, out_v, wid, r0):
        pltpu.sync_copy(fm_hbm, fm_v)

        one = np.float32(1.0)
        zero = np.float32(0.0)
        thr = np.float32(0.5)
        inf_vec = jnp.full((L,), INF, jnp.float32)
        zero_vec = jnp.full((L,), zero, jnp.float32)

        def ab_init(v, carry):
            ab_v[0, pl.ds(v * L, L)] = inf_vec
            ab_v[1, pl.ds(v * L, L)] = inf_vec
            return carry

        lax.fori_loop(0, NV, ab_init, 0)
        ab_v[2, pl.ds(0, L)] = inf_vec
        ab_v[3, pl.ds(0, L)] = inf_vec

        nseg_up = wid
        nseg_dn = (H // RPW - 1) - wid

        def scan_seg(s, fprev, side, sgn, base):
            mxp = fprev[0]
            for k in range(1, L):
                mxp = jnp.maximum(mxp, fprev[k])

            @pl.when(mxp >= BIG)
            def _():
                a = [ab_v[side, pl.ds(v * L, L)] for v in range(NV)]
                for u in range(SEG):
                    t = s * SEG + (u + 1)
                    i = base + sgn * t
                    tfv = zero_vec + t.astype(jnp.float32)
                    for v in range(NV):
                        x = fm_v[i, pl.ds(v * L, L)]
                        a[v] = jnp.minimum(a[v], jnp.where(x > thr, tfv, INF))
                for v in range(NV):
                    ab_v[side, pl.ds(v * L, L)] = a[v]
                m = a[0]
                for v in range(1, NV):
                    m = jnp.maximum(m, a[v])
                ab_v[2 + side, pl.ds(0, L)] = m

            return ab_v[2 + side, pl.ds(0, L)]

        lax.fori_loop(
            0, nseg_up, lambda s, c: scan_seg(s, c, 0, -1, r0), inf_vec)
        lax.fori_loop(
            0, nseg_dn, lambda s, c: scan_seg(s, c, 1, 1, r0 + (RPW - 1)),
            inf_vec)

        for v in range(NV):
            av = ab_v[0, pl.ds(v * L, L)]
            bv = ab_v[1, pl.ds(v * L, L)]
            xs = [fm_v[r0 + k, pl.ds(v * L, L)] for k in range(RPW)]
            fwd = []
            f = inf_vec
            for k in range(RPW):
                f = jnp.where(xs[k] > thr, zero, f + one)
                fwd.append(f)
            bw = inf_vec
            for k in range(RPW - 1, -1, -1):
                bw = jnp.where(xs[k] > thr, zero, bw + one)
                d = jnp.minimum(fwd[k], bw)
                d = jnp.minimum(d, av + np.float32(k))
                d = jnp.minimum(d, bv + np.float32(RPW - 1 - k))
                g2_v[k, pl.ds(v * L, L)] = d * d

        lane = lax.iota(jnp.int32, L).astype(jnp.float32)

        def mp_outer(v, carry):
            jvec = lane + (v * L).astype(jnp.float32)

            def chunk_min(c, accs):
                gvecs = [g2_v[r, pl.ds(c * L, L)] for r in range(RPW)]
                base = (c * L).astype(jnp.float32)
                for k in range(L):
                    diff = jvec - (base + np.float32(k))
                    pv = diff * diff
                    accs = tuple(
                        jnp.minimum(accs[r], pv + gvecs[r][k])
                        for r in range(RPW)
                    )
                return accs

            accs0 = chunk_min(
                v, tuple(jnp.full((L,), INF, jnp.float32) for _ in range(RPW))
            )

            m = accs0[0]
            for r in range(1, RPW):
                m = jnp.maximum(m, accs0[r])
            mx = m[0]
            for k in range(1, L):
                mx = jnp.maximum(mx, m[k])
            nb = jnp.int32(0)
            for d in range(1, NV):
                t = np.float32((16 * d - 15) ** 2)
                nb = nb + jnp.where(mx > t, 1, 0).astype(jnp.int32)

            accs = lax.fori_loop(
                0, jnp.minimum(nb, v),
                lambda i, a: chunk_min(v - 1 - i, a), accs0)
            accs = lax.fori_loop(
                0, jnp.minimum(nb, NV - 1 - v),
                lambda i, a: chunk_min(v + 1 + i, a), accs)
            for r in range(RPW):
                out_v[r, pl.ds(v * L, L)] = _newton_sqrt(accs[r])
            return carry

        lax.fori_loop(0, NV, mp_outer, 0)

        pltpu.sync_copy(out_v, out_hbm.at[pl.ds(r0, RPW)])

    return edt


_edt = _make_edt()


def kernel(feature_map):
    fm = feature_map.reshape(H, W)
    dist = _edt(fm)
    return jnp.broadcast_to(dist[None, None], feature_map.shape)


# speedup vs baseline: 1.0078x; 1.0078x over previous
import functools

import jax
import jax.numpy as jnp
import numpy as np
from jax import lax
from jax.experimental import pallas as pl
from jax.experimental.pallas import tpu as pltpu
from jax.experimental.pallas import tpu_sc as plsc

H = 224
W = 224
L = 16
NV = W // L
NC = 2
NS = 16
NW = NC * NS
RPW = 8
NWORK = H // RPW
SEG = 8

INF = np.float32(np.inf)
BIG = np.float32(1e30)


def _newton_sqrt(x):
    xc = jnp.where(x < BIG, jnp.maximum(x, np.float32(1.0)), np.float32(1.0))
    xr = xc
    scale = jnp.full_like(x, np.float32(1.0))
    for p in (8, 4, 2, 1):
        c = xr >= np.float32(4.0**p)
        xr = jnp.where(c, xr * np.float32(4.0 ** (-p)), xr)
        scale = jnp.where(c, scale * np.float32(2.0**p), scale)
    half = np.float32(0.5)
    y = half * (xr + np.float32(1.0))
    for _ in range(3):
        y = half * (y + xr / y)
    s = scale * y
    s = jnp.where(x < BIG, s, INF)
    return jnp.where(x == np.float32(0.0), np.float32(0.0), s)


def _make_edt():
    mesh = plsc.VectorSubcoreMesh(
        core_axis_name="c", subcore_axis_name="s",
        num_cores=NC, num_subcores=NS,
    )

    @functools.partial(
        pl.kernel,
        out_type=jax.ShapeDtypeStruct((H, W), jnp.float32),
        mesh=mesh,
        scratch_types=[
            pltpu.VMEM((H, W), jnp.float32),
            pltpu.VMEM((4, W), jnp.float32),
            pltpu.VMEM((RPW, W), jnp.float32),
            pltpu.VMEM((RPW, W), jnp.float32),
        ],
    )
    def edt(fm_hbm, out_hbm, fm_v, ab_v, g2_v, out_v):
        wid = lax.axis_index("s") * NC + lax.axis_index("c")
        r0 = wid * RPW

        @pl.when(wid < NWORK)
        def _active():
            _edt_body(fm_hbm, out_hbm, fm_v, ab_v, g2_v, out_v, wid, r0)

    def _edt_body(fm_hbm, out_hbm, fm_v, ab_v, g2_v, out_v, wid, r0):
        pltpu.sync_copy(fm_hbm, fm_v)

        one = np.float32(1.0)
        zero = np.float32(0.0)
        thr = np.float32(0.5)
        inf_vec = jnp.full((L,), INF, jnp.float32)
        zero_vec = jnp.full((L,), zero, jnp.float32)

        def ab_init(v, carry):
            ab_v[0, pl.ds(v * L, L)] = inf_vec
            ab_v[1, pl.ds(v * L, L)] = inf_vec
            return carry

        lax.fori_loop(0, NV, ab_init, 0)
        ab_v[2, pl.ds(0, L)] = inf_vec
        ab_v[3, pl.ds(0, L)] = inf_vec

        nseg_up = wid
        nseg_dn = (H // RPW - 1) - wid

        def scan_seg(s, fprev, side, sgn, base):
            mxp = fprev[0]
            for k in range(1, L):
                mxp = jnp.maximum(mxp, fprev[k])

            @pl.when(mxp >= BIG)
            def _():
                a = [ab_v[side, pl.ds(v * L, L)] for v in range(NV)]
                for u in range(SEG):
                    t = s * SEG + (u + 1)
                    i = base + sgn * t
                    tfv = zero_vec + t.astype(jnp.float32)
                    for v in range(NV):
                        x = fm_v[i, pl.ds(v * L, L)]
                        a[v] = jnp.minimum(a[v], jnp.where(x > thr, tfv, INF))
                for v in range(NV):
                    ab_v[side, pl.ds(v * L, L)] = a[v]
                m = a[0]
                for v in range(1, NV):
                    m = jnp.maximum(m, a[v])
                ab_v[2 + side, pl.ds(0, L)] = m

            return ab_v[2 + side, pl.ds(0, L)]

        lax.fori_loop(
            0, nseg_up, lambda s, c: scan_seg(s, c, 0, -1, r0), inf_vec)
        lax.fori_loop(
            0, nseg_dn, lambda s, c: scan_seg(s, c, 1, 1, r0 + (RPW - 1)),
            inf_vec)

        for v in range(NV):
            av = ab_v[0, pl.ds(v * L, L)]
            bv = ab_v[1, pl.ds(v * L, L)]
            xs = [fm_v[r0 + k, pl.ds(v * L, L)] for k in range(RPW)]
            fwd = []
            f = inf_vec
            for k in range(RPW):
                f = jnp.where(xs[k] > thr, zero, f + one)
                fwd.append(f)
            bw = inf_vec
            for k in range(RPW - 1, -1, -1):
                bw = jnp.where(xs[k] > thr, zero, bw + one)
                d = jnp.minimum(fwd[k], bw)
                d = jnp.minimum(d, av + np.float32(k))
                d = jnp.minimum(d, bv + np.float32(RPW - 1 - k))
                g2_v[k, pl.ds(v * L, L)] = d * d

        lane = lax.iota(jnp.int32, L).astype(jnp.float32)

        def mp_outer(v, carry):
            jvec = lane + (v * L).astype(jnp.float32)

            def chunk_min(c, accs):
                gvecs = [g2_v[r, pl.ds(c * L, L)] for r in range(RPW)]
                base = (c * L).astype(jnp.float32)
                for k in range(L):
                    diff = jvec - (base + np.float32(k))
                    pv = diff * diff
                    accs = tuple(
                        jnp.minimum(accs[r], pv + gvecs[r][k])
                        for r in range(RPW)
                    )
                return accs

            accs0 = chunk_min(
                v, tuple(jnp.full((L,), INF, jnp.float32) for _ in range(RPW))
            )

            m = accs0[0]
            for r in range(1, RPW):
                m = jnp.maximum(m, accs0[r])
            mx = m[0]
            for k in range(1, L):
                mx = jnp.maximum(mx, m[k])
            nb = jnp.int32(0)
            for d in range(1, NV):
                t = np.float32((16 * d - 15) ** 2)
                nb = nb + jnp.where(mx > t, 1, 0).astype(jnp.int32)

            accs = lax.fori_loop(
                0, jnp.minimum(nb, v),
                lambda i, a: chunk_min(v - 1 - i, a), accs0)
            accs = lax.fori_loop(
                0, jnp.minimum(nb, NV - 1 - v),
                lambda i, a: chunk_min(v + 1 + i, a), accs)
            for r in range(RPW):
                out_v[r, pl.ds(v * L, L)] = _newton_sqrt(accs[r])
            return carry

        lax.fori_loop(0, NV, mp_outer, 0)

        pltpu.sync_copy(out_v, out_hbm.at[pl.ds(r0, RPW)])

    return edt


_edt = _make_edt()


def kernel(feature_map):
    fm = feature_map.reshape(H, W)
    dist = _edt(fm)
    return jnp.broadcast_to(dist[None, None], feature_map.shape)
